# Initial kernel scaffold; baseline (speedup 1.0000x reference)
import jax
import jax.numpy as jnp
from jax.experimental import pallas as pl


def kernel(position_labels, pos_embedding_weight):
    raise NotImplementedError("write your pallas kernel here")


# speedup vs baseline: 3.1790x; 3.1790x over previous
import functools

import jax
import jax.numpy as jnp
from jax import lax
from jax.experimental import pallas as pl
from jax.experimental.pallas import tpu as pltpu
from jax.experimental.pallas import tpu_sc as plsc

_NUM_CORES = 2
_NUM_SUBCORES = 16
_NW = _NUM_CORES * _NUM_SUBCORES

_CHUNK = 128


def _gather_rows(table, idx_flat, n_rows, d):
    b_per_w = n_rows // _NW
    n_chunks = b_per_w // _CHUNK
    mesh = plsc.VectorSubcoreMesh(core_axis_name="c", subcore_axis_name="s")

    @functools.partial(
        pl.kernel,
        mesh=mesh,
        out_type=jax.ShapeDtypeStruct((n_rows, d), jnp.float32),
        compiler_params=pltpu.CompilerParams(use_tc_tiling_on_sc=False),
        scratch_types=[
            pltpu.VMEM((_CHUNK,), jnp.int32),
            pltpu.VMEM((_CHUNK, d), jnp.float32),
            pltpu.SemaphoreType.DMA,
        ],
    )
    def gather_kernel(table_hbm, idx_hbm, out_hbm, idx_v, rows_v, sem):
        wid = lax.axis_index("s") * _NUM_CORES + lax.axis_index("c")
        base = wid * b_per_w

        def body(i, carry):
            off = base + i * _CHUNK
            pltpu.sync_copy(idx_hbm.at[pl.ds(off, _CHUNK)], idx_v)
            pltpu.async_copy(table_hbm.at[idx_v], rows_v, sem).wait()
            pltpu.sync_copy(rows_v, out_hbm.at[pl.ds(off, _CHUNK)])
            return carry

        lax.fori_loop(0, n_chunks, body, 0)

    return gather_kernel(table, idx_flat)


def kernel(position_labels, pos_embedding_weight):
    b, s = position_labels.shape
    v, d = pos_embedding_weight.shape
    n_rows = b * s
    idx_flat = position_labels.reshape(n_rows).astype(jnp.int32)
    out = _gather_rows(pos_embedding_weight, idx_flat, n_rows, d)
    return out.reshape(b, s, d)


# speedup vs baseline: 4.1984x; 1.3207x over previous
import functools

import jax
import jax.numpy as jnp
from jax import lax
from jax.experimental import pallas as pl
from jax.experimental.pallas import tpu as pltpu
from jax.experimental.pallas import tpu_sc as plsc

_NUM_CORES = 2
_NUM_SUBCORES = 16
_NW = _NUM_CORES * _NUM_SUBCORES

_C = 128
_K = 5


def _gather_rows(table, idx_2d, n_rows, d):
    gc = _C * _K
    b_per_w = n_rows // _NW
    n_groups = b_per_w // gc
    chunks_per_w = b_per_w // _C
    mesh = plsc.VectorSubcoreMesh(core_axis_name="c", subcore_axis_name="s")

    @functools.partial(
        pl.kernel,
        mesh=mesh,
        out_type=jax.ShapeDtypeStruct((n_rows, d), jnp.float32),
        compiler_params=pltpu.CompilerParams(use_tc_tiling_on_sc=False),
        scratch_types=[
            pltpu.VMEM((2, _K, _C), jnp.int32),
            pltpu.VMEM((2, gc, d), jnp.float32),
            pltpu.SemaphoreType.DMA((2,)),
        ],
    )
    def gather_kernel(table_hbm, idx_hbm, out_hbm, idx_v, rows_v, sem):
        wid = lax.axis_index("s") * _NUM_CORES + lax.axis_index("c")
        row_base = wid * b_per_w
        chunk_base = wid * chunks_per_w

        def fire(g, buf):
            pltpu.sync_copy(idx_hbm.at[pl.ds(chunk_base + g * _K, _K)],
                            idx_v.at[buf])
            for j in range(_K):
                pltpu.async_copy(table_hbm.at[idx_v.at[buf, j]],
                                 rows_v.at[buf, pl.ds(j * _C, _C)],
                                 sem.at[buf])

        def drain(g, buf):
            pltpu.make_async_copy(table_hbm.at[pl.ds(0, gc)],
                                  rows_v.at[buf], sem.at[buf]).wait()
            pltpu.sync_copy(rows_v.at[buf],
                            out_hbm.at[pl.ds(row_base + g * gc, gc)])

        fire(0, 0)

        def body(g, carry):
            buf = lax.rem(g, 2)
            fire(g + 1, 1 - buf)
            drain(g, buf)
            return carry

        lax.fori_loop(0, n_groups - 1, body, 0)
        drain(n_groups - 1, (n_groups - 1) % 2)

    return gather_kernel(table, idx_2d)


def kernel(position_labels, pos_embedding_weight):
    b, s = position_labels.shape
    v, d = pos_embedding_weight.shape
    n_rows = b * s
    idx_2d = position_labels.reshape(n_rows // _C, _C).astype(jnp.int32)
    out = _gather_rows(pos_embedding_weight, idx_2d, n_rows, d)
    return out.reshape(b, s, d)
